# speedup vs baseline: 8.1859x; 8.1859x over previous
import functools

import jax
import jax.numpy as jnp
from jax import lax
from jax.experimental import pallas as pl
from jax.experimental.pallas import tpu as pltpu
from jax.experimental.pallas import tpu_sc as plsc

_VOCAB = 100000
_D = 128
_BATCH = 4096
_HIST = 200
_B_TOTAL = _BATCH * _HIST
_NC, _NS = 2, 16
_NW = _NC * _NS
_ROWS_PER_GATHER = 128
_K = 4
_CHUNK = _K * _ROWS_PER_GATHER
_B_PER_W = _B_TOTAL // _NW
_IDXROWS_PER_W = _B_PER_W // _ROWS_PER_GATHER
_NCHUNKS = _B_PER_W // _CHUNK

_mesh = plsc.VectorSubcoreMesh(
    core_axis_name="c", subcore_axis_name="s", num_cores=_NC, num_subcores=_NS
)


@functools.partial(
    pl.kernel,
    out_type=jax.ShapeDtypeStruct((_B_TOTAL, _D), jnp.float32),
    mesh=_mesh,
    scratch_types=[
        pltpu.VMEM((_K, _ROWS_PER_GATHER), jnp.int32),
        pltpu.VMEM((_CHUNK, _D), jnp.float32),
        pltpu.SemaphoreType.DMA,
    ],
)
def _emb_lookup(x_hbm, table_hbm, out_hbm, idx_v, rows_v, gsem):
    wid = lax.axis_index("s") * _NC + lax.axis_index("c")
    idxrow0 = wid * _IDXROWS_PER_W
    out0 = wid * _B_PER_W

    def chunk_body(ci, carry):
        pltpu.sync_copy(x_hbm.at[pl.ds(idxrow0 + ci * _K, _K)], idx_v)
        copies = [
            pltpu.async_copy(
                table_hbm.at[idx_v.at[j]],
                rows_v.at[pl.ds(j * _ROWS_PER_GATHER, _ROWS_PER_GATHER)],
                gsem,
            )
            for j in range(_K)
        ]
        for c in copies:
            c.wait()
        pltpu.sync_copy(rows_v, out_hbm.at[pl.ds(out0 + ci * _CHUNK, _CHUNK)])
        return carry

    lax.fori_loop(0, _NCHUNKS, chunk_body, 0)


def kernel(x, table):
    xr = x.astype(jnp.int32).reshape(_B_TOTAL // _ROWS_PER_GATHER, _ROWS_PER_GATHER)
    out = _emb_lookup(xr, table)
    return out.reshape(_BATCH, _HIST, _D)


# speedup vs baseline: 9.1749x; 1.1208x over previous
import functools

import jax
import jax.numpy as jnp
from jax import lax
from jax.experimental import pallas as pl
from jax.experimental.pallas import tpu as pltpu
from jax.experimental.pallas import tpu_sc as plsc

_VOCAB = 100000
_D = 128
_BATCH = 4096
_HIST = 200
_B_TOTAL = _BATCH * _HIST
_NC, _NS = 2, 16
_NW = _NC * _NS
_CHUNK = 128
_B_PER_W = _B_TOTAL // _NW
_NCH = _B_PER_W // _CHUNK
_NBUF = 4
_NSUP = _NCH // _NBUF


_mesh = plsc.VectorSubcoreMesh(
    core_axis_name="c", subcore_axis_name="s", num_cores=_NC, num_subcores=_NS
)


@functools.partial(
    pl.kernel,
    out_type=jax.ShapeDtypeStruct((_B_TOTAL, _D), jnp.float32),
    mesh=_mesh,
    scratch_types=[
        pltpu.VMEM((_NCH, _CHUNK), jnp.int32),
        [pltpu.VMEM((_CHUNK, _D), jnp.float32) for _ in range(_NBUF)],
        [pltpu.SemaphoreType.DMA for _ in range(_NBUF)],
        [pltpu.SemaphoreType.DMA for _ in range(_NBUF)],
    ],
)
def _emb_lookup(x_hbm, table_hbm, out_hbm, idx_v, rows, gsems, osems):
    wid = lax.axis_index("s") * _NC + lax.axis_index("c")
    out0 = wid * _B_PER_W

    pltpu.sync_copy(x_hbm.at[pl.ds(wid * _NCH, _NCH)], idx_v)

    def fire_gather(b, ci):
        pltpu.async_copy(table_hbm.at[idx_v.at[ci]], rows[b], gsems[b])

    def wait_gather(b):
        pltpu.make_async_copy(table_hbm.at[idx_v.at[0]], rows[b], gsems[b]).wait()

    def fire_write(b, ci):
        pltpu.async_copy(rows[b], out_hbm.at[pl.ds(out0 + ci * _CHUNK, _CHUNK)], osems[b])

    def wait_write(b):
        pltpu.make_async_copy(rows[b], out_hbm.at[pl.ds(0, _CHUNK)], osems[b]).wait()

    fire_gather(0, 0)
    fire_gather(1, 1)

    def super_body(s, carry):
        for v in range(_NBUF):
            ci = s * _NBUF + v
            b = v
            b2 = (v + 2) % _NBUF
            wait_gather(b)
            fire_write(b, ci)
            @pl.when(ci >= 2)
            def _():
                wait_write(b2)

            @pl.when(ci + 2 < _NCH)
            def _():
                fire_gather(b2, ci + 2)

        return carry

    lax.fori_loop(0, _NSUP, super_body, 0)

    wait_write((_NCH - 2) % _NBUF)
    wait_write((_NCH - 1) % _NBUF)


def kernel(x, table):
    xr = x.astype(jnp.int32).reshape(_B_TOTAL // _CHUNK, _CHUNK)
    out = _emb_lookup(xr, table)
    return out.reshape(_BATCH, _HIST, _D)


# speedup vs baseline: 9.2057x; 1.0034x over previous
import functools

import jax
import jax.numpy as jnp
from jax import lax
from jax.experimental import pallas as pl
from jax.experimental.pallas import tpu as pltpu
from jax.experimental.pallas import tpu_sc as plsc

_VOCAB = 100000
_D = 128
_BATCH = 4096
_HIST = 200
_B_TOTAL = _BATCH * _HIST
_NC, _NS = 2, 16
_NW = _NC * _NS
_G = 128
_K = 2
_CHUNK = _K * _G
_B_PER_W = _B_TOTAL // _NW
_NIDX = _B_PER_W // _G
_NCH = _B_PER_W // _CHUNK
_NBUF = 3


_mesh = plsc.VectorSubcoreMesh(
    core_axis_name="c", subcore_axis_name="s", num_cores=_NC, num_subcores=_NS
)


@functools.partial(
    pl.kernel,
    out_type=jax.ShapeDtypeStruct((_B_TOTAL, _D), jnp.float32),
    mesh=_mesh,
    scratch_types=[
        pltpu.VMEM((_NIDX, _G), jnp.int32),
        [pltpu.VMEM((_CHUNK, _D), jnp.float32) for _ in range(_NBUF)],
        [pltpu.SemaphoreType.DMA for _ in range(_NBUF)],
        [pltpu.SemaphoreType.DMA for _ in range(_NBUF)],
    ],
)
def _emb_lookup(x_hbm, table_hbm, out_hbm, idx_v, rows, gsems, osems):
    wid = lax.axis_index("s") * _NC + lax.axis_index("c")
    out0 = wid * _B_PER_W

    pltpu.sync_copy(x_hbm.at[pl.ds(wid * _NIDX, _NIDX)], idx_v)

    def fire_gathers(b, ci):
        for j in range(_K):
            pltpu.async_copy(
                table_hbm.at[idx_v.at[ci * _K + j]],
                rows[b].at[pl.ds(j * _G, _G)],
                gsems[b],
            )

    def wait_gathers(b):
        for j in range(_K):
            pltpu.make_async_copy(
                table_hbm.at[idx_v.at[0]], rows[b].at[pl.ds(j * _G, _G)], gsems[b]
            ).wait()

    def fire_write(b, ci):
        pltpu.async_copy(rows[b], out_hbm.at[pl.ds(out0 + ci * _CHUNK, _CHUNK)], osems[b])

    def wait_write(b):
        pltpu.make_async_copy(rows[b], out_hbm.at[pl.ds(0, _CHUNK)], osems[b]).wait()

    fire_gathers(0, 0)

    def super_body(s, carry):
        for v in range(_NBUF):
            ci = s * _NBUF + v
            b = v
            bn = (v + 1) % _NBUF
            wait_gathers(b)
            fire_write(b, ci)
            @pl.when(ci >= 2)
            def _():
                wait_write(bn)

            @pl.when(ci + 1 < _NCH)
            def _():
                fire_gathers(bn, ci + 1)

        return carry

    lax.fori_loop(0, _NCH // _NBUF, super_body, 0)

    ci = _NCH - 1
    b = ci % _NBUF
    wait_gathers(b)
    fire_write(b, ci)

    wait_write((_NCH - 3) % _NBUF)
    wait_write((_NCH - 2) % _NBUF)
    wait_write((_NCH - 1) % _NBUF)


def kernel(x, table):
    xr = x.astype(jnp.int32).reshape(_B_TOTAL // _G, _G)
    out = _emb_lookup(xr, table)
    return out.reshape(_BATCH, _HIST, _D)
